# baseline (speedup 1.0000x reference)
import functools

import jax
import jax.numpy as jnp
from jax import lax
from jax.experimental import pallas as pl
from jax.experimental.pallas import tpu as pltpu
from jax.experimental.pallas import tpu_sc as plsc

B = 16384
D = 128
L = 16
NC = 2
NS = 16
NW = NC * NS
BPW = B // NW
C = 64
NCHUNK = BPW // C
NBUF = 6
PP = L + 1


def _mf_body(user_hbm, item_hbm, ut_hbm, it_hbm, out_hbm,
             uidx_v, iidx_v, urows, irows, prod_v, outbuf_v, sems, isem):
    wid = lax.axis_index("s") * NC + lax.axis_index("c")
    base = wid * BPW

    ciu = pltpu.async_copy(user_hbm.at[pl.ds(base, BPW)], uidx_v, isem)
    cii = pltpu.async_copy(item_hbm.at[pl.ds(base, BPW)], iidx_v, isem)
    ciu.wait()
    cii.wait()

    inflight = [None] * NBUF

    def fire(c):
        s = c % NBUF
        cu = pltpu.async_copy(
            ut_hbm.at[uidx_v.at[pl.ds(c * C, C)]], urows.at[s], sems.at[s])
        ci = pltpu.async_copy(
            it_hbm.at[iidx_v.at[pl.ds(c * C, C)]], irows.at[s], sems.at[s])
        inflight[s] = (cu, ci)

    for c in range(min(NBUF - 1, NCHUNK)):
        fire(c)

    for c in range(NCHUNK):
        if c + NBUF - 1 < NCHUNK:
            fire(c + NBUF - 1)
        s = c % NBUF
        cu, ci = inflight[s]
        cu.wait()
        ci.wait()

        def group(g, _, c=c, s=s):
            for j in range(L):
                r = g * L + j
                acc = urows[s, r, pl.ds(0, L)] * irows[s, r, pl.ds(0, L)]
                for k in range(1, D // L):
                    acc = acc + (urows[s, r, pl.ds(k * L, L)]
                                 * irows[s, r, pl.ds(k * L, L)])
                prod_v[j, pl.ds(0, L)] = acc
            rowidx = lax.broadcasted_iota(jnp.int32, (L,), 0)
            vec = plsc.load_gather(prod_v, [rowidx, jnp.zeros((L,), jnp.int32)])
            for l in range(1, L):
                vec = vec + plsc.load_gather(
                    prod_v, [rowidx, jnp.full((L,), l, jnp.int32)])
            vec = 1.0 / (1.0 + jnp.exp(-vec))
            outbuf_v[pl.ds(c * C + g * L, L)] = vec
            return 0

        lax.fori_loop(0, C // L, group, 0)

    pltpu.sync_copy(outbuf_v, out_hbm.at[pl.ds(base, BPW)])


@jax.jit
def kernel(user, item, user_table, item_table):
    mesh = plsc.VectorSubcoreMesh(
        core_axis_name="c", subcore_axis_name="s",
        num_cores=NC, num_subcores=NS)
    run = pl.kernel(
        _mf_body,
        out_type=jax.ShapeDtypeStruct((B,), jnp.float32),
        mesh=mesh,
        scratch_types=[
            pltpu.VMEM((BPW,), jnp.int32),
            pltpu.VMEM((BPW,), jnp.int32),
            pltpu.VMEM((NBUF, C, D), jnp.float32),
            pltpu.VMEM((NBUF, C, D), jnp.float32),
            pltpu.VMEM((L, PP), jnp.float32),
            pltpu.VMEM((BPW,), jnp.float32),
            pltpu.SemaphoreType.DMA((NBUF,)),
            pltpu.SemaphoreType.DMA,
        ],
        compiler_params=pltpu.CompilerParams(needs_layout_passes=False),
    )
    return run(user.astype(jnp.int32), item.astype(jnp.int32),
               user_table, item_table)


# speedup vs baseline: 1.0810x; 1.0810x over previous
import functools

import jax
import jax.numpy as jnp
from jax import lax
from jax.experimental import pallas as pl
from jax.experimental.pallas import tpu as pltpu
from jax.experimental.pallas import tpu_sc as plsc

B = 16384
D = 128
L = 16
NC = 2
NS = 16
NW = NC * NS
BPW = B // NW
C = 128
NCHUNK = BPW // C
NBUF = 3
PP = L + 1


def _mf_body(user_hbm, item_hbm, ut_hbm, it_hbm, out_hbm,
             uidx_v, iidx_v, urows, irows, prod_v, outbuf_v,
             sems, isem, osem):
    wid = lax.axis_index("s") * NC + lax.axis_index("c")
    base = wid * BPW

    c0u = pltpu.async_copy(user_hbm.at[pl.ds(base, C)],
                           uidx_v.at[pl.ds(0, C)], isem)
    c0i = pltpu.async_copy(item_hbm.at[pl.ds(base, C)],
                           iidx_v.at[pl.ds(0, C)], isem)
    rest = BPW - C
    cru = pltpu.async_copy(user_hbm.at[pl.ds(base + C, rest)],
                           uidx_v.at[pl.ds(C, rest)], isem)
    cri = pltpu.async_copy(item_hbm.at[pl.ds(base + C, rest)],
                           iidx_v.at[pl.ds(C, rest)], isem)
    c0u.wait()
    c0i.wait()

    inflight = [None] * NBUF
    outcopies = []

    def fire(c):
        s = c % NBUF
        cu = pltpu.async_copy(
            ut_hbm.at[uidx_v.at[pl.ds(c * C, C)]], urows.at[s], sems.at[s])
        ci = pltpu.async_copy(
            it_hbm.at[iidx_v.at[pl.ds(c * C, C)]], irows.at[s], sems.at[s])
        inflight[s] = (cu, ci)

    fire(0)
    cru.wait()
    cri.wait()
    for c in range(1, min(NBUF - 1, NCHUNK)):
        fire(c)

    for c in range(NCHUNK):
        if c + NBUF - 1 < NCHUNK:
            fire(c + NBUF - 1)
        s = c % NBUF
        cu, ci = inflight[s]
        cu.wait()
        ci.wait()

        def group(g, _, c=c, s=s):
            for j in range(L):
                r = g * L + j
                acc = urows[s, r, pl.ds(0, L)] * irows[s, r, pl.ds(0, L)]
                for k in range(1, D // L):
                    acc = acc + (urows[s, r, pl.ds(k * L, L)]
                                 * irows[s, r, pl.ds(k * L, L)])
                prod_v[j, pl.ds(0, L)] = acc
            rowidx = lax.broadcasted_iota(jnp.int32, (L,), 0)
            vec = plsc.load_gather(prod_v, [rowidx, jnp.zeros((L,), jnp.int32)])
            for l in range(1, L):
                vec = vec + plsc.load_gather(
                    prod_v, [rowidx, jnp.full((L,), l, jnp.int32)])
            vec = 1.0 / (1.0 + jnp.exp(-vec))
            outbuf_v[pl.ds(c * C + g * L, L)] = vec
            return 0

        lax.fori_loop(0, C // L, group, 0)
        outcopies.append(pltpu.async_copy(
            outbuf_v.at[pl.ds(c * C, C)],
            out_hbm.at[pl.ds(base + c * C, C)], osem))

    for oc in outcopies:
        oc.wait()


@jax.jit
def kernel(user, item, user_table, item_table):
    mesh = plsc.VectorSubcoreMesh(
        core_axis_name="c", subcore_axis_name="s",
        num_cores=NC, num_subcores=NS)
    run = pl.kernel(
        _mf_body,
        out_type=jax.ShapeDtypeStruct((B,), jnp.float32),
        mesh=mesh,
        scratch_types=[
            pltpu.VMEM((BPW,), jnp.int32),
            pltpu.VMEM((BPW,), jnp.int32),
            pltpu.VMEM((NBUF, C, D), jnp.float32),
            pltpu.VMEM((NBUF, C, D), jnp.float32),
            pltpu.VMEM((L, PP), jnp.float32),
            pltpu.VMEM((BPW,), jnp.float32),
            pltpu.SemaphoreType.DMA((NBUF,)),
            pltpu.SemaphoreType.DMA,
            pltpu.SemaphoreType.DMA,
        ],
        compiler_params=pltpu.CompilerParams(needs_layout_passes=False),
    )
    return run(user.astype(jnp.int32), item.astype(jnp.int32),
               user_table, item_table)


# speedup vs baseline: 1.0833x; 1.0021x over previous
import jax
import jax.numpy as jnp
from jax import lax
from jax.experimental import pallas as pl
from jax.experimental.pallas import tpu as pltpu
from jax.experimental.pallas import tpu_sc as plsc

B = 16384
D = 128
L = 16
NC = 2
NS = 16
NW = NC * NS
BPW = B // NW
C = 128
NCHUNK = BPW // C
NBUF = 3
PP = L + 1


def _mf_body(user_hbm, item_hbm, ut_hbm, it_hbm, out_hbm,
             uidx_v, iidx_v, urows, irows, prod_v, outbuf_v,
             sems, isem, osem):
    wid = lax.axis_index("s") * NC + lax.axis_index("c")
    base = wid * BPW

    c0u = pltpu.async_copy(user_hbm.at[pl.ds(base, C)],
                           uidx_v.at[pl.ds(0, C)], isem)
    c0i = pltpu.async_copy(item_hbm.at[pl.ds(base, C)],
                           iidx_v.at[pl.ds(0, C)], isem)
    rest = BPW - C
    cru = pltpu.async_copy(user_hbm.at[pl.ds(base + C, rest)],
                           uidx_v.at[pl.ds(C, rest)], isem)
    cri = pltpu.async_copy(item_hbm.at[pl.ds(base + C, rest)],
                           iidx_v.at[pl.ds(C, rest)], isem)
    c0u.wait()
    c0i.wait()

    inflight = [None] * NBUF
    outcopies = []

    def fire(c):
        s = c % NBUF
        cu = pltpu.async_copy(
            ut_hbm.at[uidx_v.at[pl.ds(c * C, C)]], urows.at[s], sems.at[s])
        ci = pltpu.async_copy(
            it_hbm.at[iidx_v.at[pl.ds(c * C, C)]], irows.at[s], sems.at[s])
        inflight[s] = (cu, ci)

    fire(0)
    cru.wait()
    cri.wait()
    for c in range(1, min(NBUF - 1, NCHUNK)):
        fire(c)

    for c in range(NCHUNK):
        if c + NBUF - 1 < NCHUNK:
            fire(c + NBUF - 1)
        s = c % NBUF
        cu, ci = inflight[s]
        cu.wait()
        ci.wait()

        def group(g, _, c=c, s=s):
            for j in range(L):
                r = g * L + j
                acc = urows[s, r, pl.ds(0, L)] * irows[s, r, pl.ds(0, L)]
                for k in range(1, D // L):
                    acc = acc + (urows[s, r, pl.ds(k * L, L)]
                                 * irows[s, r, pl.ds(k * L, L)])
                prod_v[j, pl.ds(0, L)] = acc
            rowidx = lax.broadcasted_iota(jnp.int32, (L,), 0)
            vec = plsc.load_gather(prod_v, [rowidx, jnp.zeros((L,), jnp.int32)])
            for l in range(1, L):
                vec = vec + plsc.load_gather(
                    prod_v, [rowidx, jnp.full((L,), l, jnp.int32)])
            vec = 1.0 / (1.0 + jnp.exp(-vec))
            outbuf_v[pl.ds(c * C + g * L, L)] = vec
            return 0

        lax.fori_loop(0, C // L, group, 0)
        outcopies.append(pltpu.async_copy(
            outbuf_v.at[pl.ds(c * C, C)],
            out_hbm.at[pl.ds(base + c * C, C)], osem))

    for oc in outcopies:
        oc.wait()


@jax.jit
def kernel(user, item, user_table, item_table):
    mesh = plsc.VectorSubcoreMesh(
        core_axis_name="c", subcore_axis_name="s",
        num_cores=NC, num_subcores=NS)
    run = pl.kernel(
        _mf_body,
        out_type=jax.ShapeDtypeStruct((B,), jnp.float32),
        mesh=mesh,
        scratch_types=[
            pltpu.VMEM((BPW,), jnp.int32),
            pltpu.VMEM((BPW,), jnp.int32),
            pltpu.VMEM((NBUF, C, D), jnp.float32),
            pltpu.VMEM((NBUF, C, D), jnp.float32),
            pltpu.VMEM((L, PP), jnp.float32),
            pltpu.VMEM((BPW,), jnp.float32),
            pltpu.SemaphoreType.DMA((NBUF,)),
            pltpu.SemaphoreType.DMA,
            pltpu.SemaphoreType.DMA,
        ],
        compiler_params=pltpu.CompilerParams(needs_layout_passes=False),
    )
    return run(user.astype(jnp.int32), item.astype(jnp.int32),
               user_table, item_table)
